# Initial kernel scaffold; baseline (speedup 1.0000x reference)
import jax
import jax.numpy as jnp
from jax.experimental import pallas as pl


def kernel(x, lut):
    raise NotImplementedError("write your pallas kernel here")


# speedup vs baseline: 2.4075x; 2.4075x over previous
import functools
import math

import jax
import jax.numpy as jnp
from jax import lax
from jax.experimental import pallas as pl
from jax.experimental.pallas import tpu as pltpu
from jax.experimental.pallas import tpu_sc as plsc

_VOCAB = 1000
_D = 128
_B = 4096 * 50
_NC = 2
_NS = 16
_NW = _NC * _NS
_BPW = _B // _NW
_CH = 128
_NCH = _BPW // _CH
_SCALE = math.sqrt(float(_D))

_TBL = _VOCAB * _D
_TPW = _TBL // _NW

_MESH = plsc.VectorSubcoreMesh(core_axis_name="c", subcore_axis_name="s")


@functools.partial(
    pl.kernel,
    out_type=jax.ShapeDtypeStruct((_TBL,), jnp.float32),
    mesh=_MESH,
    scratch_types=[pltpu.VMEM((_TPW,), jnp.float32)],
)
def _scale_table(lut_hbm, out_hbm, buf):
    wid = lax.axis_index("s") * _NC + lax.axis_index("c")
    base = wid * _TPW
    pltpu.sync_copy(lut_hbm.at[pl.ds(base, _TPW)], buf)

    def body(i, carry):
        sl = pl.ds(i * 16, 16)
        buf[sl] = buf[sl] * _SCALE
        return carry

    lax.fori_loop(0, _TPW // 16, body, 0)
    pltpu.sync_copy(buf, out_hbm.at[pl.ds(base, _TPW)])


@functools.partial(
    pl.kernel,
    out_type=jax.ShapeDtypeStruct((_B, _D), jnp.float32),
    mesh=_MESH,
    scratch_types=[
        pltpu.VMEM((_NCH, _CH), jnp.int32),
        pltpu.VMEM((_CH, _D), jnp.float32),
        pltpu.SemaphoreType.DMA,
    ],
)
def _gather(lut_hbm, idx_hbm, out_hbm, idx_v, rows_v, gsem):
    wid = lax.axis_index("s") * _NC + lax.axis_index("c")
    pltpu.sync_copy(idx_hbm.at[wid], idx_v)
    out_base = wid * _BPW

    def body(ci, carry):
        pltpu.async_copy(lut_hbm.at[idx_v.at[ci]], rows_v, gsem).wait()
        pltpu.sync_copy(rows_v, out_hbm.at[pl.ds(out_base + ci * _CH, _CH)])
        return carry

    lax.fori_loop(0, _NCH, body, 0)


def kernel(x, lut):
    lut_scaled = _scale_table(lut.reshape(_TBL)).reshape(_VOCAB, _D)
    idx = x.reshape(_NW, _NCH, _CH)
    out = _gather(lut_scaled, idx)
    return out.reshape(x.shape[0], x.shape[1], _D)


# speedup vs baseline: 2.4946x; 1.0362x over previous
import functools
import math

import jax
import jax.numpy as jnp
from jax import lax
from jax.experimental import pallas as pl
from jax.experimental.pallas import tpu as pltpu
from jax.experimental.pallas import tpu_sc as plsc

_VOCAB = 1000
_D = 128
_B = 4096 * 50
_NC = 2
_NS = 16
_NW = _NC * _NS
_BPW = _B // _NW
_CH = 128
_NCH = _BPW // _CH
_SCALE = math.sqrt(float(_D))

_TBL = _VOCAB * _D
_TPW = _TBL // _NW

_MESH = plsc.VectorSubcoreMesh(core_axis_name="c", subcore_axis_name="s")


@functools.partial(
    pl.kernel,
    out_type=jax.ShapeDtypeStruct((_TBL,), jnp.float32),
    mesh=_MESH,
    scratch_types=[pltpu.VMEM((_TPW,), jnp.float32)],
)
def _scale_table(lut_hbm, out_hbm, buf):
    wid = lax.axis_index("s") * _NC + lax.axis_index("c")
    base = wid * _TPW
    pltpu.sync_copy(lut_hbm.at[pl.ds(base, _TPW)], buf)

    def body(i, carry):
        sl = pl.ds(i * 16, 16)
        buf[sl] = buf[sl] * _SCALE
        return carry

    lax.fori_loop(0, _TPW // 16, body, 0)
    pltpu.sync_copy(buf, out_hbm.at[pl.ds(base, _TPW)])


_NB = 5
_NGRP = _NCH // _NB


@functools.partial(
    pl.kernel,
    out_type=jax.ShapeDtypeStruct((_B, _D), jnp.float32),
    mesh=_MESH,
    scratch_types=[
        pltpu.VMEM((_NCH, _CH), jnp.int32),
        pltpu.VMEM((_NB, _CH, _D), jnp.float32),
        [pltpu.SemaphoreType.DMA] * _NB,
        [pltpu.SemaphoreType.DMA] * _NB,
    ],
)
def _gather(lut_hbm, idx_hbm, out_hbm, idx_v, rows_v, gsems, ssems):
    wid = lax.axis_index("s") * _NC + lax.axis_index("c")
    pltpu.sync_copy(idx_hbm.at[wid], idx_v)
    out_base = wid * _BPW

    def gather_start(ci, b):
        pltpu.async_copy(lut_hbm.at[idx_v.at[ci]], rows_v.at[b], gsems[b])

    def gather_wait(b):
        pltpu.make_async_copy(lut_hbm.at[pl.ds(0, _CH)], rows_v.at[b],
                              gsems[b]).wait()

    def store_start(ci, b):
        return pltpu.async_copy(
            rows_v.at[b], out_hbm.at[pl.ds(out_base + ci * _CH, _CH)],
            ssems[b])

    for b in range(_NB):
        gather_start(b, b)

    def body(g, carry):
        base = g * _NB
        descs = []
        for b in range(_NB):
            gather_wait(b)
            descs.append(store_start(base + b, b))
        for b in range(_NB):
            descs[b].wait()
            gather_start(base + _NB + b, b)
        return carry

    lax.fori_loop(0, _NGRP - 1, body, 0)

    base = (_NGRP - 1) * _NB
    descs = []
    for b in range(_NB):
        gather_wait(b)
        descs.append(store_start(base + b, b))
    for b in range(_NB):
        descs[b].wait()


def kernel(x, lut):
    lut_scaled = _scale_table(lut.reshape(_TBL)).reshape(_VOCAB, _D)
    idx = x.reshape(_NW, _NCH, _CH)
    out = _gather(lut_scaled, idx)
    return out.reshape(x.shape[0], x.shape[1], _D)


# speedup vs baseline: 3.9485x; 1.5828x over previous
import functools
import math

import jax
import jax.numpy as jnp
from jax import lax
from jax.experimental import pallas as pl
from jax.experimental.pallas import tpu as pltpu
from jax.experimental.pallas import tpu_sc as plsc

_VOCAB = 1000
_D = 128
_BATCH = 4096
_HIST = 50
_NC = 2
_NS = 16
_NW = _NC * _NS
_BPW = _BATCH // _NW
_SCALE = math.sqrt(float(_D))

_SROWS = 40
_SWORK = _VOCAB // _SROWS

_NB = 4
_NGRP = _BPW // _NB

_MESH = plsc.VectorSubcoreMesh(core_axis_name="c", subcore_axis_name="s")
_PARAMS = pltpu.CompilerParams(use_tc_tiling_on_sc=True)


@functools.partial(
    pl.kernel,
    out_type=jax.ShapeDtypeStruct((_VOCAB, _D), jnp.float32),
    mesh=_MESH,
    compiler_params=_PARAMS,
    scratch_types=[pltpu.VMEM((_SROWS, _D), jnp.float32)],
)
def _scale_table(lut_hbm, out_hbm, buf):
    wid = lax.axis_index("s") * _NC + lax.axis_index("c")

    @pl.when(wid < _SWORK)
    def _():
        base = wid * _SROWS
        pltpu.sync_copy(lut_hbm.at[pl.ds(base, _SROWS)], buf)

        def body(i, carry):
            r = i // (_D // 16)
            c = (i % (_D // 16)) * 16
            buf[r, pl.ds(c, 16)] = buf[r, pl.ds(c, 16)] * _SCALE
            return carry

        lax.fori_loop(0, _SROWS * _D // 16, body, 0)
        pltpu.sync_copy(buf, out_hbm.at[pl.ds(base, _SROWS)])


@functools.partial(
    pl.kernel,
    out_type=jax.ShapeDtypeStruct((_BATCH, _HIST, _D), jnp.float32),
    mesh=_MESH,
    compiler_params=_PARAMS,
    scratch_types=[
        pltpu.VMEM((_BPW, _HIST), jnp.int32),
        pltpu.VMEM((_NB, _HIST, _D), jnp.float32),
        [pltpu.SemaphoreType.DMA] * _NB,
        [pltpu.SemaphoreType.DMA] * _NB,
    ],
)
def _gather(lut_hbm, idx_hbm, out_hbm, idx_v, rows_v, gsems, ssems):
    wid = lax.axis_index("s") * _NC + lax.axis_index("c")
    out_base = wid * _BPW
    pltpu.sync_copy(idx_hbm.at[pl.ds(out_base, _BPW)], idx_v)

    def gather_start(bi, b):
        pltpu.async_copy(lut_hbm.at[idx_v.at[bi]], rows_v.at[b], gsems[b])

    def gather_wait(bi, b):
        pltpu.make_async_copy(lut_hbm.at[idx_v.at[bi]], rows_v.at[b],
                              gsems[b]).wait()

    def store_start(bi, b):
        return pltpu.async_copy(rows_v.at[b], out_hbm.at[out_base + bi],
                                ssems[b])

    for b in range(_NB):
        gather_start(b, b)

    def body(g, carry):
        base = g * _NB
        descs = []
        for b in range(_NB):
            gather_wait(base + b, b)
            descs.append(store_start(base + b, b))
        for b in range(_NB):
            descs[b].wait()
            gather_start(base + _NB + b, b)
        return carry

    lax.fori_loop(0, _NGRP - 1, body, 0)

    base = (_NGRP - 1) * _NB
    descs = []
    for b in range(_NB):
        gather_wait(base + b, b)
        descs.append(store_start(base + b, b))
    for b in range(_NB):
        descs[b].wait()


def kernel(x, lut):
    lut_scaled = _scale_table(lut)
    return _gather(lut_scaled, x)


# speedup vs baseline: 5.7995x; 1.4688x over previous
import functools
import math

import jax
import jax.numpy as jnp
from jax import lax
from jax.experimental import pallas as pl
from jax.experimental.pallas import tpu as pltpu
from jax.experimental.pallas import tpu_sc as plsc

_VOCAB = 1000
_D = 128
_BATCH = 4096
_HIST = 50
_NC = 2
_NS = 16
_NW = _NC * _NS
_BPW = _BATCH // _NW
_SCALE = math.sqrt(float(_D))

_SROWS = 40
_SWORK = _VOCAB // _SROWS

_NB = 5
_NGRP = _HIST // _NB

_MESH = plsc.VectorSubcoreMesh(core_axis_name="c", subcore_axis_name="s")
_PARAMS = pltpu.CompilerParams(use_tc_tiling_on_sc=True)


@functools.partial(
    pl.kernel,
    out_type=jax.ShapeDtypeStruct((_VOCAB, _D), jnp.float32),
    mesh=_MESH,
    compiler_params=_PARAMS,
    scratch_types=[pltpu.VMEM((_SROWS, _D), jnp.float32)],
)
def _scale_table(lut_hbm, out_hbm, buf):
    wid = lax.axis_index("s") * _NC + lax.axis_index("c")

    @pl.when(wid < _SWORK)
    def _():
        base = wid * _SROWS
        pltpu.sync_copy(lut_hbm.at[pl.ds(base, _SROWS)], buf)

        def body(i, carry):
            r = i // (_D // 16)
            c = (i % (_D // 16)) * 16
            buf[r, pl.ds(c, 16)] = buf[r, pl.ds(c, 16)] * _SCALE
            return carry

        lax.fori_loop(0, _SROWS * _D // 16, body, 0)
        pltpu.sync_copy(buf, out_hbm.at[pl.ds(base, _SROWS)])


@functools.partial(
    pl.kernel,
    out_type=jax.ShapeDtypeStruct((_HIST, _BATCH, _D), jnp.float32),
    mesh=_MESH,
    compiler_params=_PARAMS,
    scratch_types=[
        pltpu.VMEM((_HIST, _BPW), jnp.int32),
        pltpu.VMEM((_NB, _BPW, _D), jnp.float32),
        [pltpu.SemaphoreType.DMA] * _NB,
        [pltpu.SemaphoreType.DMA] * _NB,
    ],
)
def _gather(lut_hbm, idx_hbm, out_hbm, idx_v, rows_v, gsems, ssems):
    wid = lax.axis_index("s") * _NC + lax.axis_index("c")
    out_base = wid * _BPW
    pltpu.sync_copy(idx_hbm.at[wid], idx_v)

    def gather_start(h, b):
        pltpu.async_copy(lut_hbm.at[idx_v.at[h]], rows_v.at[b], gsems[b])

    def gather_wait(h, b):
        pltpu.make_async_copy(lut_hbm.at[idx_v.at[h]], rows_v.at[b],
                              gsems[b]).wait()

    def store_start(h, b):
        return pltpu.async_copy(
            rows_v.at[b], out_hbm.at[h, pl.ds(out_base, _BPW)], ssems[b])

    for b in range(_NB):
        gather_start(b, b)

    def body(g, carry):
        base = g * _NB
        descs = []
        for b in range(_NB):
            gather_wait(base + b, b)
            descs.append(store_start(base + b, b))
        for b in range(_NB):
            descs[b].wait()
            gather_start(base + _NB + b, b)
        return carry

    lax.fori_loop(0, _NGRP - 1, body, 0)

    base = (_NGRP - 1) * _NB
    descs = []
    for b in range(_NB):
        gather_wait(base + b, b)
        descs.append(store_start(base + b, b))
    for b in range(_NB):
        descs[b].wait()


def kernel(x, lut):
    lut_scaled = _scale_table(lut)
    idx_t = x.reshape(_NW, _BPW, _HIST).transpose(0, 2, 1)
    out_t = _gather(lut_scaled, idx_t)
    return out_t.transpose(1, 0, 2)


# speedup vs baseline: 12.3785x; 2.1344x over previous
import functools
import math

import jax
import jax.numpy as jnp
from jax import lax
from jax.experimental import pallas as pl
from jax.experimental.pallas import tpu as pltpu
from jax.experimental.pallas import tpu_sc as plsc

_VOCAB = 1000
_D = 128
_BATCH = 4096
_HIST = 50
_NC = 2
_NS = 16
_NW = _NC * _NS
_BPW = _BATCH // _NW
_SCALE = math.sqrt(float(_D))

_SROWS = 40
_SWORK = _VOCAB // _SROWS

_NB = 5
_NGRP = _HIST // _NB

_MESH = plsc.VectorSubcoreMesh(core_axis_name="c", subcore_axis_name="s")
_PARAMS = pltpu.CompilerParams(use_tc_tiling_on_sc=True)


@functools.partial(
    pl.kernel,
    out_type=jax.ShapeDtypeStruct((_VOCAB, _D), jnp.float32),
    mesh=_MESH,
    compiler_params=_PARAMS,
    scratch_types=[pltpu.VMEM((_SROWS, _D), jnp.float32)],
)
def _scale_table(lut_hbm, out_hbm, buf):
    wid = lax.axis_index("s") * _NC + lax.axis_index("c")

    @pl.when(wid < _SWORK)
    def _():
        base = wid * _SROWS
        pltpu.sync_copy(lut_hbm.at[pl.ds(base, _SROWS)], buf)

        def body(i, carry):
            r = i // (_D // 16)
            c = (i % (_D // 16)) * 16
            buf[r, pl.ds(c, 16)] = buf[r, pl.ds(c, 16)] * _SCALE
            return carry

        lax.fori_loop(0, _SROWS * _D // 16, body, 0)
        pltpu.sync_copy(buf, out_hbm.at[pl.ds(base, _SROWS)])


@functools.partial(
    pl.kernel,
    out_type=jax.ShapeDtypeStruct((_HIST, _BATCH, _D), jnp.float32),
    mesh=_MESH,
    compiler_params=_PARAMS,
    scratch_types=[
        pltpu.VMEM((_HIST, _BPW), jnp.int32),
        pltpu.VMEM((_NB, _BPW, _D), jnp.float32),
        pltpu.VMEM_SHARED((_VOCAB, _D), jnp.float32),
        [pltpu.SemaphoreType.DMA] * _NB,
        [pltpu.SemaphoreType.DMA] * _NB,
    ],
)
def _gather(lut_hbm, idx_hbm, out_hbm, idx_v, rows_v, table_sh, gsems, ssems):
    wid = lax.axis_index("s") * _NC + lax.axis_index("c")
    out_base = wid * _BPW

    @pl.when(lax.axis_index("s") == 0)
    def _():
        pltpu.sync_copy(lut_hbm, table_sh)

    pltpu.sync_copy(idx_hbm.at[wid], idx_v)
    plsc.subcore_barrier()

    def gather_start(h, b):
        pltpu.async_copy(table_sh.at[idx_v.at[h]], rows_v.at[b], gsems[b])

    def gather_wait(h, b):
        pltpu.make_async_copy(table_sh.at[idx_v.at[h]], rows_v.at[b],
                              gsems[b]).wait()

    def store_start(h, b):
        return pltpu.async_copy(
            rows_v.at[b], out_hbm.at[h, pl.ds(out_base, _BPW)], ssems[b])

    for b in range(_NB):
        gather_start(b, b)

    def body(g, carry):
        base = g * _NB
        descs = []
        for b in range(_NB):
            gather_wait(base + b, b)
            descs.append(store_start(base + b, b))
        for b in range(_NB):
            descs[b].wait()
            gather_start(base + _NB + b, b)
        return carry

    lax.fori_loop(0, _NGRP - 1, body, 0)

    base = (_NGRP - 1) * _NB
    descs = []
    for b in range(_NB):
        gather_wait(base + b, b)
        descs.append(store_start(base + b, b))
    for b in range(_NB):
        descs[b].wait()


def kernel(x, lut):
    lut_scaled = _scale_table(lut)
    idx_t = x.reshape(_NW, _BPW, _HIST).transpose(0, 2, 1)
    out_t = _gather(lut_scaled, idx_t)
    return out_t.transpose(1, 0, 2)


# speedup vs baseline: 13.6590x; 1.1034x over previous
import functools
import math

import jax
import jax.numpy as jnp
from jax import lax
from jax.experimental import pallas as pl
from jax.experimental.pallas import tpu as pltpu
from jax.experimental.pallas import tpu_sc as plsc

_VOCAB = 1000
_D = 128
_BATCH = 4096
_HIST = 50
_NC = 2
_NS = 16
_NW = _NC * _NS
_BPW = _BATCH // _NW
_SCALE = math.sqrt(float(_D))

_SROWS = 64
_SLAST = _VOCAB - (_NS - 1) * _SROWS

_NB = 5
_NGRP = _HIST // _NB

_MESH = plsc.VectorSubcoreMesh(core_axis_name="c", subcore_axis_name="s")
_PARAMS = pltpu.CompilerParams(use_tc_tiling_on_sc=True)


@functools.partial(
    pl.kernel,
    out_type=jax.ShapeDtypeStruct((_HIST, _BATCH, _D), jnp.float32),
    mesh=_MESH,
    compiler_params=_PARAMS,
    scratch_types=[
        pltpu.VMEM((_HIST, _BPW), jnp.int32),
        pltpu.VMEM((_NB, _BPW, _D), jnp.float32),
        pltpu.VMEM_SHARED((_VOCAB, _D), jnp.float32),
        pltpu.SemaphoreType.DMA,
        [pltpu.SemaphoreType.DMA] * _NB,
        [pltpu.SemaphoreType.DMA] * _NB,
    ],
)
def _embed(lut_hbm, idx_hbm, out_hbm, idx_v, rows_v, table_sh, isem,
           gsems, ssems):
    sid = lax.axis_index("s")
    wid = sid * _NC + lax.axis_index("c")
    out_base = wid * _BPW

    idx_desc = pltpu.async_copy(idx_hbm.at[wid], idx_v, isem)

    def scale_slice(rows, base):
        stage = rows_v.at[0, pl.ds(0, rows)]
        pltpu.sync_copy(lut_hbm.at[pl.ds(base, rows)], stage)

        def body(r, carry):
            for c in range(_D // 16):
                sl = pl.ds(c * 16, 16)
                stage[r, sl] = stage[r, sl] * _SCALE
            return carry

        lax.fori_loop(0, rows, body, 0)
        pltpu.sync_copy(stage, table_sh.at[pl.ds(base, rows)])

    @pl.when(sid < _NS - 1)
    def _():
        scale_slice(_SROWS, sid * _SROWS)

    @pl.when(sid == _NS - 1)
    def _():
        scale_slice(_SLAST, (_NS - 1) * _SROWS)

    idx_desc.wait()
    plsc.subcore_barrier()

    def gather_start(h, b):
        pltpu.async_copy(table_sh.at[idx_v.at[h]], rows_v.at[b], gsems[b])

    def gather_wait(h, b):
        pltpu.make_async_copy(table_sh.at[idx_v.at[h]], rows_v.at[b],
                              gsems[b]).wait()

    def store_start(h, b):
        return pltpu.async_copy(
            rows_v.at[b], out_hbm.at[h, pl.ds(out_base, _BPW)], ssems[b])

    for b in range(_NB):
        gather_start(b, b)

    def body(g, carry):
        base = g * _NB
        descs = []
        for b in range(_NB):
            gather_wait(base + b, b)
            descs.append(store_start(base + b, b))
        for b in range(_NB):
            descs[b].wait()
            gather_start(base + _NB + b, b)
        return carry

    lax.fori_loop(0, _NGRP - 1, body, 0)

    base = (_NGRP - 1) * _NB
    descs = []
    for b in range(_NB):
        gather_wait(base + b, b)
        descs.append(store_start(base + b, b))
    for b in range(_NB):
        descs[b].wait()


def kernel(x, lut):
    idx_t = x.reshape(_NW, _BPW, _HIST).transpose(0, 2, 1)
    out_t = _embed(lut, idx_t)
    return out_t.transpose(1, 0, 2)
